# baseline (speedup 1.0000x reference)
import jax
import jax.numpy as jnp
from jax import lax
from jax.experimental import pallas as pl
from jax.experimental.pallas import tpu as pltpu
from jax.experimental.pallas import tpu_sc as plsc

N_NODES = 10000
N_PAD = 10240
N_EDGES = 320000
NC, NS = 2, 16
CH = 128
CPW = 160
E_PAD = NS * CPW * CH
D_HID = 32
HF = D_HID // NC
STRIPE = N_PAD // NS


def _sc_mesh():
    return plsc.VectorSubcoreMesh(core_axis_name="c", subcore_axis_name="s")


_SC_PARAMS = pltpu.CompilerParams(use_tc_tiling_on_sc=False)


def _mega_body(h1h_hbm, src_hbm, dst_hbm,
               out_hbm, tab1_hbm, tab2_hbm,
               sidx_v, didx_v, rows0_v, rows1_v, rows2_v, rows3_v,
               accT, hT, dsT, dT, zT, z1, ones_v, deg_sh, acc_sh,
               gsem0, gsem1, gsem2, gsem3, ssem0, ssem1, ssem2, ssem3):
    c = lax.axis_index("c")
    s = lax.axis_index("s")
    base = s * STRIPE
    bufs = (rows0_v, rows1_v, rows2_v, rows3_v)
    gsems = (gsem0, gsem1, gsem2, gsem3)
    ssems = (ssem0, ssem1, ssem2, ssem3)

    z16 = jnp.zeros((16,), jnp.float32)
    for i in range(STRIPE // 16):
        z1[pl.ds(i * 16, 16)] = z16
    one16 = jnp.full((16,), 1.0, jnp.float32)
    for i in range(CH // 16):
        ones_v[pl.ds(i * 16, 16)] = one16

    def zrow(r, carry):
        zT[r] = jnp.zeros((HF,), jnp.float32)
        return carry

    lax.fori_loop(0, STRIPE, zrow, 0)
    pltpu.async_copy(src_hbm.at[s], sidx_v, gsem0).wait()
    pltpu.async_copy(dst_hbm.at[s], didx_v, gsem0).wait()
    pltpu.sync_copy(zT, acc_sh.at[pl.ds(base, STRIPE)])
    pltpu.sync_copy(z1, deg_sh.at[pl.ds(base, STRIPE)])
    plsc.subcore_barrier()

    for t in range(4):
        pltpu.async_copy(ones_v, deg_sh.at[didx_v.at[t]], ssems[t], add=True)

    def deg4(k, carry):
        b4 = 4 * k
        for t in range(4):
            pltpu.make_async_copy(ones_v, deg_sh.at[didx_v.at[b4 + t]],
                                  ssems[t]).wait()
            pltpu.async_copy(ones_v, deg_sh.at[didx_v.at[b4 + 4 + t]],
                             ssems[t], add=True)
        return carry

    lax.fori_loop(0, CPW // 4 - 1, deg4, 0)
    for t in range(4):
        pltpu.make_async_copy(ones_v, deg_sh.at[didx_v.at[CPW - 4 + t]],
                              ssems[t]).wait()
    plsc.subcore_barrier()

    pltpu.sync_copy(deg_sh.at[pl.ds(base, STRIPE)], dsT)

    def rsq(i, carry):
        v = dsT[pl.ds(i * 16, 16)] + 1.0
        yi = jnp.int32(0x5F3759DF) - (lax.bitcast_convert_type(v, jnp.int32) >> 1)
        y = lax.bitcast_convert_type(yi, jnp.float32)
        y = y * (1.5 - 0.5 * v * y * y)
        y = y * (1.5 - 0.5 * v * y * y)
        y = y * (1.5 - 0.5 * v * y * y)
        for rr in range(16):
            dT[i * 16 + rr] = jnp.broadcast_to(y[rr], (HF,))
        return carry

    lax.fori_loop(0, STRIPE // 16, rsq, 0)

    pltpu.sync_copy(h1h_hbm.at[c, pl.ds(base, STRIPE)], hT)

    def scale(r, carry):
        hT[r] = hT[r] * dT[r]
        return carry

    lax.fori_loop(0, STRIPE, scale, 0)
    pltpu.sync_copy(hT, tab1_hbm.at[c, pl.ds(base, STRIPE)])
    plsc.subcore_barrier()

    def prop(tab_view):
        def gather(j, slot):
            pltpu.async_copy(tab_view.at[sidx_v.at[j]], bufs[slot],
                             gsems[slot])

        def wait_gather(j, slot):
            pltpu.make_async_copy(tab_view.at[sidx_v.at[j]], bufs[slot],
                                  gsems[slot]).wait()

        def scatter(j, slot):
            pltpu.async_copy(bufs[slot], acc_sh.at[didx_v.at[j]],
                             ssems[slot], add=True)

        def wait_scatter(j, slot):
            pltpu.make_async_copy(bufs[slot], acc_sh.at[didx_v.at[j]],
                                  ssems[slot]).wait()

        gather(0, 0)
        gather(1, 1)
        for j in (0, 1):
            wait_gather(j, j)
            scatter(j, j)
            gather(j + 2, (j + 2) % 4)

        def body4(k, carry):
            base4 = 4 * k + 2
            for t in range(4):
                j = base4 + t
                slot = (2 + t) % 4
                wait_gather(j, slot)
                scatter(j, slot)
                wait_scatter(j - 2, (slot + 2) % 4)
                gather(j + 2, (slot + 2) % 4)
            return carry

        lax.fori_loop(0, (CPW - 4) // 4, body4, 0)

        for j in (CPW - 2, CPW - 1):
            wait_gather(j, j % 4)
            scatter(j, j % 4)
            wait_scatter(j - 2, (j - 2) % 4)
        wait_scatter(CPW - 2, (CPW - 2) % 4)
        wait_scatter(CPW - 1, (CPW - 1) % 4)

    prop(tab1_hbm.at[c])
    plsc.subcore_barrier()

    pltpu.sync_copy(acc_sh.at[pl.ds(base, STRIPE)], accT)

    def mid(r, carry):
        d = dT[r]
        t = (accT[r] + hT[r]) * d
        e = jnp.exp(2.0 * t)
        hT[r] = (1.0 - 2.0 / (e + 1.0)) * d
        return carry

    lax.fori_loop(0, STRIPE, mid, 0)
    pltpu.sync_copy(hT, tab2_hbm.at[c, pl.ds(base, STRIPE)])
    pltpu.sync_copy(zT, acc_sh.at[pl.ds(base, STRIPE)])
    plsc.subcore_barrier()

    prop(tab2_hbm.at[c])
    plsc.subcore_barrier()

    pltpu.sync_copy(acc_sh.at[pl.ds(base, STRIPE)], accT)

    def fin(r, carry):
        accT[r] = (accT[r] + hT[r]) * dT[r]
        return carry

    lax.fori_loop(0, STRIPE, fin, 0)
    pltpu.sync_copy(accT, out_hbm.at[c, pl.ds(base, STRIPE)])


def _make_mega_kernel():
    return pl.kernel(
        _mega_body,
        out_type=(jax.ShapeDtypeStruct((NC, N_PAD, HF), jnp.float32),
                  jax.ShapeDtypeStruct((NC, N_PAD, HF), jnp.float32),
                  jax.ShapeDtypeStruct((NC, N_PAD, HF), jnp.float32)),
        mesh=_sc_mesh(),
        scratch_types=[
            pltpu.VMEM((CPW, CH), jnp.int32),
            pltpu.VMEM((CPW, CH), jnp.int32),
            pltpu.VMEM((CH, HF), jnp.float32),
            pltpu.VMEM((CH, HF), jnp.float32),
            pltpu.VMEM((CH, HF), jnp.float32),
            pltpu.VMEM((CH, HF), jnp.float32),
            pltpu.VMEM((STRIPE, HF), jnp.float32),
            pltpu.VMEM((STRIPE, HF), jnp.float32),
            pltpu.VMEM((STRIPE,), jnp.float32),
            pltpu.VMEM((STRIPE, HF), jnp.float32),
            pltpu.VMEM((STRIPE, HF), jnp.float32),
            pltpu.VMEM((STRIPE,), jnp.float32),
            pltpu.VMEM((CH,), jnp.float32),
            pltpu.VMEM_SHARED((N_PAD,), jnp.float32),
            pltpu.VMEM_SHARED((N_PAD, HF), jnp.float32),
        ] + [pltpu.SemaphoreType.DMA] * 8,
        compiler_params=_SC_PARAMS,
    )


BRA = 1000
BRZ = 1024


def _h1_body(x_ref, w_ref, o_ref):
    o_ref[...] = jnp.dot(x_ref[...], w_ref[...],
                         preferred_element_type=jnp.float32)


def _zmat_body(p_ref, w2_ref, o_ref):
    o_ref[...] = jnp.dot(p_ref[...], w2_ref[...],
                         preferred_element_type=jnp.float32)


def _h1_call(x, W1):
    return pl.pallas_call(
        _h1_body,
        grid=(N_NODES // BRA,),
        in_specs=[pl.BlockSpec((BRA, 128), lambda i: (i, 0)),
                  pl.BlockSpec((128, D_HID), lambda i: (0, 0))],
        out_specs=pl.BlockSpec((BRA, D_HID), lambda i: (i, 0)),
        out_shape=jax.ShapeDtypeStruct((N_NODES, D_HID), jnp.float32),
    )(x, W1)


def _zmat_call(p2, W2p):
    return pl.pallas_call(
        _zmat_body,
        grid=(N_PAD // BRZ,),
        in_specs=[pl.BlockSpec((BRZ, D_HID), lambda i: (i, 0)),
                  pl.BlockSpec((D_HID, 128), lambda i: (0, 0))],
        out_specs=pl.BlockSpec((BRZ, 128), lambda i: (i, 0)),
        out_shape=jax.ShapeDtypeStruct((N_PAD, 128), jnp.float32),
    )(p2, W2p)


def kernel(x, edge_index, W1, W2):
    n = x.shape[0]
    dummy = jnp.full((E_PAD - N_EDGES,), n, dtype=jnp.int32)
    srcp = jnp.concatenate([edge_index[0], dummy]).reshape(NS, CPW, CH)
    dstp = jnp.concatenate([edge_index[1], dummy]).reshape(NS, CPW, CH)
    W2p = jnp.pad(W2, ((0, 0), (0, 128 - W2.shape[1])))

    h1s = _h1_call(x, W1)
    h1h = jnp.pad(h1s, ((0, N_PAD - n), (0, 0))).reshape(
        N_PAD, NC, HF).transpose(1, 0, 2)
    p2h, _tab1, _tab2 = _make_mega_kernel()(h1h, srcp, dstp)
    p2 = p2h.transpose(1, 0, 2).reshape(N_PAD, D_HID)
    zp = _zmat_call(p2, W2p)
    z = zp[:n, :W2.shape[1]]
    return (z, z)


# speedup vs baseline: 1.0654x; 1.0654x over previous
import jax
import jax.numpy as jnp
from jax import lax
from jax.experimental import pallas as pl
from jax.experimental.pallas import tpu as pltpu
from jax.experimental.pallas import tpu_sc as plsc

N_NODES = 10000
N_PAD = 10240
N_EDGES = 320000
NC, NS = 2, 16
CH = 128
CPW = 160
E_PAD = NS * CPW * CH
D_HID = 32
HF = D_HID // NC
STRIPE = N_PAD // NS


def _sc_mesh():
    return plsc.VectorSubcoreMesh(core_axis_name="c", subcore_axis_name="s")


_SC_PARAMS = pltpu.CompilerParams(use_tc_tiling_on_sc=False)


DEG_CPW = CPW // NC


def _deg_body(dst_hbm, idx_v, ones_v, z1, deg_sh, sem, out_hbm):
    c = lax.axis_index("c")
    s = lax.axis_index("s")
    z16 = jnp.zeros((16,), jnp.float32)
    one16 = jnp.full((16,), 1.0, jnp.float32)
    for i in range(STRIPE // 16):
        z1[pl.ds(i * 16, 16)] = z16
    for i in range(CH // 16):
        ones_v[pl.ds(i * 16, 16)] = one16
    pltpu.sync_copy(z1, deg_sh.at[pl.ds(s * STRIPE, STRIPE)])
    pltpu.async_copy(dst_hbm.at[s, pl.ds(c * DEG_CPW, DEG_CPW)], idx_v,
                     sem).wait()
    plsc.subcore_barrier()

    def body(j, carry):
        pltpu.sync_copy(ones_v, deg_sh.at[idx_v.at[j]], add=True)
        return carry

    lax.fori_loop(0, DEG_CPW, body, 0)
    plsc.subcore_barrier()
    pltpu.sync_copy(deg_sh.at[pl.ds(s * STRIPE, STRIPE)],
                    out_hbm.at[c, pl.ds(s * STRIPE, STRIPE)])


def _make_deg_kernel():
    def body(dst_hbm, out_hbm, idx_v, ones_v, z1, deg_sh, sem):
        _deg_body(dst_hbm, idx_v, ones_v, z1, deg_sh, sem, out_hbm)

    return pl.kernel(
        body,
        out_type=jax.ShapeDtypeStruct((NC, N_PAD), jnp.float32),
        mesh=_sc_mesh(),
        scratch_types=[
            pltpu.VMEM((DEG_CPW, CH), jnp.int32),
            pltpu.VMEM((CH,), jnp.float32),
            pltpu.VMEM((STRIPE,), jnp.float32),
            pltpu.VMEM_SHARED((N_PAD,), jnp.float32),
            pltpu.SemaphoreType.DMA,
        ],
        compiler_params=_SC_PARAMS,
    )


def _mega_body(h1h_hbm, src_hbm, dst_hbm, degp_hbm,
               out_hbm, tab1_hbm, tab2_hbm,
               sidx_v, didx_v, rows0_v, rows1_v, rows2_v, rows3_v,
               accT, hT, dsT, d2T, dT, zT,
               acc_sh,
               gsem0, gsem1, gsem2, gsem3, ssem0, ssem1, ssem2, ssem3):
    c = lax.axis_index("c")
    s = lax.axis_index("s")
    base = s * STRIPE
    bufs = (rows0_v, rows1_v, rows2_v, rows3_v)
    gsems = (gsem0, gsem1, gsem2, gsem3)
    ssems = (ssem0, ssem1, ssem2, ssem3)

    def zrow(r, carry):
        zT[r] = jnp.zeros((HF,), jnp.float32)
        return carry

    lax.fori_loop(0, STRIPE, zrow, 0)
    pltpu.async_copy(src_hbm.at[s], sidx_v, gsem0).wait()
    pltpu.async_copy(dst_hbm.at[s], didx_v, gsem0).wait()
    pltpu.sync_copy(zT, acc_sh.at[pl.ds(base, STRIPE)])
    pltpu.sync_copy(degp_hbm.at[0, pl.ds(base, STRIPE)], dsT)
    pltpu.sync_copy(degp_hbm.at[1, pl.ds(base, STRIPE)], d2T)

    def rsq(i, carry):
        v = dsT[pl.ds(i * 16, 16)] + d2T[pl.ds(i * 16, 16)] + 1.0
        yi = jnp.int32(0x5F3759DF) - (lax.bitcast_convert_type(v, jnp.int32) >> 1)
        y = lax.bitcast_convert_type(yi, jnp.float32)
        y = y * (1.5 - 0.5 * v * y * y)
        y = y * (1.5 - 0.5 * v * y * y)
        y = y * (1.5 - 0.5 * v * y * y)
        for rr in range(16):
            dT[i * 16 + rr] = jnp.broadcast_to(y[rr], (HF,))
        return carry

    lax.fori_loop(0, STRIPE // 16, rsq, 0)

    pltpu.sync_copy(h1h_hbm.at[c, pl.ds(base, STRIPE)], hT)

    def scale(r, carry):
        hT[r] = hT[r] * dT[r]
        return carry

    lax.fori_loop(0, STRIPE, scale, 0)
    pltpu.sync_copy(hT, tab1_hbm.at[c, pl.ds(base, STRIPE)])
    plsc.subcore_barrier()

    def prop(tab_view):
        def gather(j, slot):
            pltpu.async_copy(tab_view.at[sidx_v.at[j]], bufs[slot],
                             gsems[slot])

        def wait_gather(j, slot):
            pltpu.make_async_copy(tab_view.at[sidx_v.at[j]], bufs[slot],
                                  gsems[slot]).wait()

        def scatter(j, slot):
            pltpu.async_copy(bufs[slot], acc_sh.at[didx_v.at[j]],
                             ssems[slot], add=True)

        def wait_scatter(j, slot):
            pltpu.make_async_copy(bufs[slot], acc_sh.at[didx_v.at[j]],
                                  ssems[slot]).wait()

        gather(0, 0)
        gather(1, 1)
        for j in (0, 1):
            wait_gather(j, j)
            scatter(j, j)
            gather(j + 2, (j + 2) % 4)

        def body4(k, carry):
            base4 = 4 * k + 2
            for t in range(4):
                j = base4 + t
                slot = (2 + t) % 4
                wait_gather(j, slot)
                scatter(j, slot)
                wait_scatter(j - 2, (slot + 2) % 4)
                gather(j + 2, (slot + 2) % 4)
            return carry

        lax.fori_loop(0, (CPW - 4) // 4, body4, 0)

        for j in (CPW - 2, CPW - 1):
            wait_gather(j, j % 4)
            scatter(j, j % 4)
            wait_scatter(j - 2, (j - 2) % 4)
        wait_scatter(CPW - 2, (CPW - 2) % 4)
        wait_scatter(CPW - 1, (CPW - 1) % 4)

    prop(tab1_hbm.at[c])
    plsc.subcore_barrier()

    pltpu.sync_copy(acc_sh.at[pl.ds(base, STRIPE)], accT)

    def mid(r, carry):
        d = dT[r]
        t = (accT[r] + hT[r]) * d
        e = jnp.exp(2.0 * t)
        hT[r] = (1.0 - 2.0 / (e + 1.0)) * d
        return carry

    lax.fori_loop(0, STRIPE, mid, 0)
    pltpu.sync_copy(hT, tab2_hbm.at[c, pl.ds(base, STRIPE)])
    pltpu.sync_copy(zT, acc_sh.at[pl.ds(base, STRIPE)])
    plsc.subcore_barrier()

    prop(tab2_hbm.at[c])
    plsc.subcore_barrier()

    pltpu.sync_copy(acc_sh.at[pl.ds(base, STRIPE)], accT)

    def fin(r, carry):
        accT[r] = (accT[r] + hT[r]) * dT[r]
        return carry

    lax.fori_loop(0, STRIPE, fin, 0)
    pltpu.sync_copy(accT, out_hbm.at[c, pl.ds(base, STRIPE)])


def _make_mega_kernel():
    return pl.kernel(
        _mega_body,
        out_type=(jax.ShapeDtypeStruct((NC, N_PAD, HF), jnp.float32),
                  jax.ShapeDtypeStruct((NC, N_PAD, HF), jnp.float32),
                  jax.ShapeDtypeStruct((NC, N_PAD, HF), jnp.float32)),
        mesh=_sc_mesh(),
        scratch_types=[
            pltpu.VMEM((CPW, CH), jnp.int32),
            pltpu.VMEM((CPW, CH), jnp.int32),
            pltpu.VMEM((CH, HF), jnp.float32),
            pltpu.VMEM((CH, HF), jnp.float32),
            pltpu.VMEM((CH, HF), jnp.float32),
            pltpu.VMEM((CH, HF), jnp.float32),
            pltpu.VMEM((STRIPE, HF), jnp.float32),
            pltpu.VMEM((STRIPE, HF), jnp.float32),
            pltpu.VMEM((STRIPE,), jnp.float32),
            pltpu.VMEM((STRIPE,), jnp.float32),
            pltpu.VMEM((STRIPE, HF), jnp.float32),
            pltpu.VMEM((STRIPE, HF), jnp.float32),
            pltpu.VMEM_SHARED((N_PAD, HF), jnp.float32),
        ] + [pltpu.SemaphoreType.DMA] * 8,
        compiler_params=_SC_PARAMS,
    )


BRA = 1000
BRZ = 1024


def _h1_body(x_ref, w_ref, o_ref):
    o_ref[...] = jnp.dot(x_ref[...], w_ref[...],
                         preferred_element_type=jnp.float32)


def _zmat_body(p_ref, w2_ref, o_ref):
    o_ref[...] = jnp.dot(p_ref[...], w2_ref[...],
                         preferred_element_type=jnp.float32)


def _h1_call(x, W1):
    return pl.pallas_call(
        _h1_body,
        grid=(N_NODES // BRA,),
        in_specs=[pl.BlockSpec((BRA, 128), lambda i: (i, 0)),
                  pl.BlockSpec((128, D_HID), lambda i: (0, 0))],
        out_specs=pl.BlockSpec((BRA, D_HID), lambda i: (i, 0)),
        out_shape=jax.ShapeDtypeStruct((N_NODES, D_HID), jnp.float32),
    )(x, W1)


def _zmat_call(p2, W2p):
    return pl.pallas_call(
        _zmat_body,
        grid=(N_PAD // BRZ,),
        in_specs=[pl.BlockSpec((BRZ, D_HID), lambda i: (i, 0)),
                  pl.BlockSpec((D_HID, 128), lambda i: (0, 0))],
        out_specs=pl.BlockSpec((BRZ, 128), lambda i: (i, 0)),
        out_shape=jax.ShapeDtypeStruct((N_PAD, 128), jnp.float32),
    )(p2, W2p)


def kernel(x, edge_index, W1, W2):
    n = x.shape[0]
    dummy = jnp.full((E_PAD - N_EDGES,), n, dtype=jnp.int32)
    srcp = jnp.concatenate([edge_index[0], dummy]).reshape(NS, CPW, CH)
    dstp = jnp.concatenate([edge_index[1], dummy]).reshape(NS, CPW, CH)
    W2p = jnp.pad(W2, ((0, 0), (0, 128 - W2.shape[1])))

    deg_part = _make_deg_kernel()(dstp)
    h1s = _h1_call(x, W1)
    h1h = jnp.pad(h1s, ((0, N_PAD - n), (0, 0))).reshape(
        N_PAD, NC, HF).transpose(1, 0, 2)
    p2h, _tab1, _tab2 = _make_mega_kernel()(h1h, srcp, dstp, deg_part)
    p2 = p2h.transpose(1, 0, 2).reshape(N_PAD, D_HID)
    zp = _zmat_call(p2, W2p)
    z = zp[:n, :W2.shape[1]]
    return (z, z)


# speedup vs baseline: 1.1978x; 1.1243x over previous
import jax
import jax.numpy as jnp
from jax import lax
from jax.experimental import pallas as pl
from jax.experimental.pallas import tpu as pltpu
from jax.experimental.pallas import tpu_sc as plsc

N_NODES = 10000
N_PAD = 10240
N_EDGES = 320000
NC, NS = 2, 16
CH = 512
CPW = 40
E_PAD = NS * CPW * CH
D_HID = 32
HF = D_HID // NC
STRIPE = N_PAD // NS


def _sc_mesh():
    return plsc.VectorSubcoreMesh(core_axis_name="c", subcore_axis_name="s")


_SC_PARAMS = pltpu.CompilerParams(use_tc_tiling_on_sc=False)


DEG_CPW = CPW // NC


def _deg_body(dst_hbm, idx_v, ones_v, z1, deg_sh, sem, out_hbm):
    c = lax.axis_index("c")
    s = lax.axis_index("s")
    z16 = jnp.zeros((16,), jnp.float32)
    one16 = jnp.full((16,), 1.0, jnp.float32)
    for i in range(STRIPE // 16):
        z1[pl.ds(i * 16, 16)] = z16
    for i in range(CH // 16):
        ones_v[pl.ds(i * 16, 16)] = one16
    pltpu.sync_copy(z1, deg_sh.at[pl.ds(s * STRIPE, STRIPE)])
    pltpu.async_copy(dst_hbm.at[s, pl.ds(c * DEG_CPW, DEG_CPW)], idx_v,
                     sem).wait()
    plsc.subcore_barrier()

    def body(j, carry):
        pltpu.sync_copy(ones_v, deg_sh.at[idx_v.at[j]], add=True)
        return carry

    lax.fori_loop(0, DEG_CPW, body, 0)
    plsc.subcore_barrier()
    pltpu.sync_copy(deg_sh.at[pl.ds(s * STRIPE, STRIPE)],
                    out_hbm.at[c, pl.ds(s * STRIPE, STRIPE)])


def _make_deg_kernel():
    def body(dst_hbm, out_hbm, idx_v, ones_v, z1, deg_sh, sem):
        _deg_body(dst_hbm, idx_v, ones_v, z1, deg_sh, sem, out_hbm)

    return pl.kernel(
        body,
        out_type=jax.ShapeDtypeStruct((NC, N_PAD), jnp.float32),
        mesh=_sc_mesh(),
        scratch_types=[
            pltpu.VMEM((DEG_CPW, CH), jnp.int32),
            pltpu.VMEM((CH,), jnp.float32),
            pltpu.VMEM((STRIPE,), jnp.float32),
            pltpu.VMEM_SHARED((N_PAD,), jnp.float32),
            pltpu.SemaphoreType.DMA,
        ],
        compiler_params=_SC_PARAMS,
    )


def _mega_body(h1h_hbm, src_hbm, dst_hbm, degp_hbm,
               out_hbm, tab1_hbm, tab2_hbm,
               sidx_v, didx_v, rows0_v, rows1_v, rows2_v, rows3_v,
               accT, hT, dsT, d2T, dT, zT,
               acc_sh,
               gsem0, gsem1, gsem2, gsem3, ssem0, ssem1, ssem2, ssem3):
    c = lax.axis_index("c")
    s = lax.axis_index("s")
    base = s * STRIPE
    bufs = (rows0_v, rows1_v, rows2_v, rows3_v)
    gsems = (gsem0, gsem1, gsem2, gsem3)
    ssems = (ssem0, ssem1, ssem2, ssem3)

    def zrow(r, carry):
        zT[r] = jnp.zeros((HF,), jnp.float32)
        return carry

    lax.fori_loop(0, STRIPE, zrow, 0)
    pltpu.async_copy(src_hbm.at[s], sidx_v, gsem0).wait()
    pltpu.async_copy(dst_hbm.at[s], didx_v, gsem0).wait()
    pltpu.sync_copy(zT, acc_sh.at[pl.ds(base, STRIPE)])
    pltpu.sync_copy(degp_hbm.at[0, pl.ds(base, STRIPE)], dsT)
    pltpu.sync_copy(degp_hbm.at[1, pl.ds(base, STRIPE)], d2T)

    def rsq(i, carry):
        v = dsT[pl.ds(i * 16, 16)] + d2T[pl.ds(i * 16, 16)] + 1.0
        yi = jnp.int32(0x5F3759DF) - (lax.bitcast_convert_type(v, jnp.int32) >> 1)
        y = lax.bitcast_convert_type(yi, jnp.float32)
        y = y * (1.5 - 0.5 * v * y * y)
        y = y * (1.5 - 0.5 * v * y * y)
        y = y * (1.5 - 0.5 * v * y * y)
        for rr in range(16):
            dT[i * 16 + rr] = jnp.broadcast_to(y[rr], (HF,))
        return carry

    lax.fori_loop(0, STRIPE // 16, rsq, 0)

    pltpu.sync_copy(h1h_hbm.at[c, pl.ds(base, STRIPE)], hT)

    def scale(r, carry):
        hT[r] = hT[r] * dT[r]
        return carry

    lax.fori_loop(0, STRIPE, scale, 0)
    pltpu.sync_copy(hT, tab1_hbm.at[c, pl.ds(base, STRIPE)])
    plsc.subcore_barrier()

    def prop(tab_view):
        def gather(j, slot):
            pltpu.async_copy(tab_view.at[sidx_v.at[j]], bufs[slot],
                             gsems[slot])

        def wait_gather(j, slot):
            pltpu.make_async_copy(tab_view.at[sidx_v.at[j]], bufs[slot],
                                  gsems[slot]).wait()

        def scatter(j, slot):
            pltpu.async_copy(bufs[slot], acc_sh.at[didx_v.at[j]],
                             ssems[slot], add=True)

        def wait_scatter(j, slot):
            pltpu.make_async_copy(bufs[slot], acc_sh.at[didx_v.at[j]],
                                  ssems[slot]).wait()

        gather(0, 0)
        gather(1, 1)
        for j in (0, 1):
            wait_gather(j, j)
            scatter(j, j)
            gather(j + 2, (j + 2) % 4)

        def body4(k, carry):
            base4 = 4 * k + 2
            for t in range(4):
                j = base4 + t
                slot = (2 + t) % 4
                wait_gather(j, slot)
                scatter(j, slot)
                wait_scatter(j - 2, (slot + 2) % 4)
                gather(j + 2, (slot + 2) % 4)
            return carry

        lax.fori_loop(0, (CPW - 4) // 4, body4, 0)

        for j in (CPW - 2, CPW - 1):
            wait_gather(j, j % 4)
            scatter(j, j % 4)
            wait_scatter(j - 2, (j - 2) % 4)
        wait_scatter(CPW - 2, (CPW - 2) % 4)
        wait_scatter(CPW - 1, (CPW - 1) % 4)

    prop(tab1_hbm.at[c])
    plsc.subcore_barrier()

    pltpu.sync_copy(acc_sh.at[pl.ds(base, STRIPE)], accT)

    def mid(r, carry):
        d = dT[r]
        t = (accT[r] + hT[r]) * d
        e = jnp.exp(2.0 * t)
        hT[r] = (1.0 - 2.0 / (e + 1.0)) * d
        return carry

    lax.fori_loop(0, STRIPE, mid, 0)
    pltpu.sync_copy(hT, tab2_hbm.at[c, pl.ds(base, STRIPE)])
    pltpu.sync_copy(zT, acc_sh.at[pl.ds(base, STRIPE)])
    plsc.subcore_barrier()

    prop(tab2_hbm.at[c])
    plsc.subcore_barrier()

    pltpu.sync_copy(acc_sh.at[pl.ds(base, STRIPE)], accT)

    def fin(r, carry):
        accT[r] = (accT[r] + hT[r]) * dT[r]
        return carry

    lax.fori_loop(0, STRIPE, fin, 0)
    pltpu.sync_copy(accT, out_hbm.at[c, pl.ds(base, STRIPE)])


def _make_mega_kernel():
    return pl.kernel(
        _mega_body,
        out_type=(jax.ShapeDtypeStruct((NC, N_PAD, HF), jnp.float32),
                  jax.ShapeDtypeStruct((NC, N_PAD, HF), jnp.float32),
                  jax.ShapeDtypeStruct((NC, N_PAD, HF), jnp.float32)),
        mesh=_sc_mesh(),
        scratch_types=[
            pltpu.VMEM((CPW, CH), jnp.int32),
            pltpu.VMEM((CPW, CH), jnp.int32),
            pltpu.VMEM((CH, HF), jnp.float32),
            pltpu.VMEM((CH, HF), jnp.float32),
            pltpu.VMEM((CH, HF), jnp.float32),
            pltpu.VMEM((CH, HF), jnp.float32),
            pltpu.VMEM((STRIPE, HF), jnp.float32),
            pltpu.VMEM((STRIPE, HF), jnp.float32),
            pltpu.VMEM((STRIPE,), jnp.float32),
            pltpu.VMEM((STRIPE,), jnp.float32),
            pltpu.VMEM((STRIPE, HF), jnp.float32),
            pltpu.VMEM((STRIPE, HF), jnp.float32),
            pltpu.VMEM_SHARED((N_PAD, HF), jnp.float32),
        ] + [pltpu.SemaphoreType.DMA] * 8,
        compiler_params=_SC_PARAMS,
    )


BRA = 1000
BRZ = 1024


def _h1_body(x_ref, w_ref, o_ref):
    o_ref[...] = jnp.dot(x_ref[...], w_ref[...],
                         preferred_element_type=jnp.float32)


def _zmat_body(p_ref, w2_ref, o_ref):
    o_ref[...] = jnp.dot(p_ref[...], w2_ref[...],
                         preferred_element_type=jnp.float32)


def _h1_call(x, W1):
    return pl.pallas_call(
        _h1_body,
        grid=(N_NODES // BRA,),
        in_specs=[pl.BlockSpec((BRA, 128), lambda i: (i, 0)),
                  pl.BlockSpec((128, D_HID), lambda i: (0, 0))],
        out_specs=pl.BlockSpec((BRA, D_HID), lambda i: (i, 0)),
        out_shape=jax.ShapeDtypeStruct((N_NODES, D_HID), jnp.float32),
    )(x, W1)


def _zmat_call(p2, W2p):
    return pl.pallas_call(
        _zmat_body,
        grid=(N_PAD // BRZ,),
        in_specs=[pl.BlockSpec((BRZ, D_HID), lambda i: (i, 0)),
                  pl.BlockSpec((D_HID, 128), lambda i: (0, 0))],
        out_specs=pl.BlockSpec((BRZ, 128), lambda i: (i, 0)),
        out_shape=jax.ShapeDtypeStruct((N_PAD, 128), jnp.float32),
    )(p2, W2p)


def kernel(x, edge_index, W1, W2):
    n = x.shape[0]
    dummy = jnp.full((E_PAD - N_EDGES,), n, dtype=jnp.int32)
    srcp = jnp.concatenate([edge_index[0], dummy]).reshape(NS, CPW, CH)
    dstp = jnp.concatenate([edge_index[1], dummy]).reshape(NS, CPW, CH)
    W2p = jnp.pad(W2, ((0, 0), (0, 128 - W2.shape[1])))

    deg_part = _make_deg_kernel()(dstp)
    h1s = _h1_call(x, W1)
    h1h = jnp.pad(h1s, ((0, N_PAD - n), (0, 0))).reshape(
        N_PAD, NC, HF).transpose(1, 0, 2)
    p2h, _tab1, _tab2 = _make_mega_kernel()(h1h, srcp, dstp, deg_part)
    p2 = p2h.transpose(1, 0, 2).reshape(N_PAD, D_HID)
    zp = _zmat_call(p2, W2p)
    z = zp[:n, :W2.shape[1]]
    return (z, z)
